# baseline (speedup 1.0000x reference)
import functools

import jax
import jax.numpy as jnp
from jax import lax
from jax.experimental import pallas as pl

B = 4
L = 4096
D = 256
H = 8
DH = D // H
FF = 1024
TOPK = 1000
KP = 1024
RT = 32
KT = 512
NKT = L // KT


def _topk_positions_kernel(sal_ref, prob_ref, pm_ref):
    s = prob_ref[...] * jax.nn.sigmoid(sal_ref[...])
    bits = lax.bitcast_convert_type(s, jnp.int32)
    lo0 = jnp.zeros((B, 1, 1), jnp.int32)
    hi0 = jnp.full((B, 1, 1), 0x7F7FFFFF, jnp.int32)

    def body(_, carry):
        lo, hi = carry
        mid = lo + (hi - lo + 1) // 2
        cnt = jnp.sum((bits >= mid).astype(jnp.int32), axis=(1, 2), keepdims=True)
        ok = cnt >= TOPK
        return jnp.where(ok, mid, lo), jnp.where(ok, hi, mid - 1)

    lo, _ = lax.fori_loop(0, 31, body, (lo0, hi0))
    mask = bits >= lo
    m = mask.astype(jnp.float32)
    c0 = lax.broadcasted_iota(jnp.int32, (128, 128), 0)
    c1 = lax.broadcasted_iota(jnp.int32, (128, 128), 1)
    upper = (c0 <= c1).astype(jnp.float32)
    r0 = lax.broadcasted_iota(jnp.int32, (RT, RT), 0)
    r1 = lax.broadcasted_iota(jnp.int32, (RT, RT), 1)
    strict = (r0 < r1).astype(jnp.float32)
    pin = lax.dot_general(m, upper, (((2,), (0,)), ((), ())),
                          precision=lax.Precision.HIGHEST)
    rs = jnp.sum(m, axis=2)
    ro = lax.dot_general(rs, strict, (((1,), (0,)), ((), ())),
                         precision=lax.Precision.HIGHEST)
    p = pin + ro[:, :, None]
    pm_ref[...] = jnp.where(mask, p.astype(jnp.int32), 0)


def _attn_kernel(pm_ref, q_ref, pos_ref, ln_s_ref, ln_b_ref,
                 wq_ref, wk_ref, wv_ref, wo_ref, out_ref):
    pmv = pm_ref[0, 0, :]
    hp = lax.Precision.HIGHEST

    selq = jnp.zeros((KP, D), jnp.float32)
    selpos = jnp.zeros((KP, D), jnp.float32)
    jr = lax.broadcasted_iota(jnp.int32, (KP, KT), 0) + 1
    for t in range(NKT):
        pm_t = pmv[t * KT:(t + 1) * KT]
        s_t = (pm_t[None, :] == jr).astype(jnp.float32)
        selq = selq + jnp.dot(s_t, q_ref[0, t * KT:(t + 1) * KT, :],
                              precision=hp)
        selpos = selpos + jnp.dot(s_t, pos_ref[0, t * KT:(t + 1) * KT, :],
                                  precision=hp)

    x = selq
    mu = jnp.mean(x, axis=1, keepdims=True)
    var = jnp.mean((x - mu) ** 2, axis=1, keepdims=True)
    x2 = (x - mu) * lax.rsqrt(var + 1e-5) * ln_s_ref[...] + ln_b_ref[...]
    qk_in = x2 + selpos
    q = jnp.dot(qk_in, wq_ref[...], precision=hp)
    k = jnp.dot(qk_in, wk_ref[...], precision=hp)
    v = jnp.dot(x2, wv_ref[...], precision=hp)

    scale = 1.0 / (DH ** 0.5)
    kcol = lax.broadcasted_iota(jnp.int32, (KP, KP), 1)
    kmask = jnp.where(kcol >= TOPK, -1e9, 0.0)
    ohs = []
    for h in range(H):
        qh = q[:, h * DH:(h + 1) * DH]
        kh = k[:, h * DH:(h + 1) * DH]
        vh = v[:, h * DH:(h + 1) * DH]
        logits = lax.dot_general(qh, kh, (((1,), (1,)), ((), ())),
                                 precision=hp) * scale + kmask
        mx = jnp.max(logits, axis=1, keepdims=True)
        e = jnp.exp(logits - mx)
        a = e / jnp.sum(e, axis=1, keepdims=True)
        ohs.append(jnp.dot(a, vh, precision=hp))
    oh = jnp.concatenate(ohs, axis=1)
    attn = x + jnp.dot(oh, wo_ref[...], precision=hp)

    for t in range(NKT):
        pm_t = pmv[t * KT:(t + 1) * KT]
        s_t = (pm_t[None, :] == jr).astype(jnp.float32)
        contrib = lax.dot_general(s_t, attn, (((0,), (0,)), ((), ())),
                                  precision=hp)
        keep = (pm_t == 0).astype(jnp.float32)[:, None]
        out_ref[0, t * KT:(t + 1) * KT, :] = (
            q_ref[0, t * KT:(t + 1) * KT, :] * keep + contrib)


def _ffn_kernel(x_ref, ln_s_ref, ln_b_ref, w1_ref, b1_ref, w2_ref, b2_ref,
                out_ref):
    x = x_ref[...]
    mu = jnp.mean(x, axis=1, keepdims=True)
    var = jnp.mean((x - mu) ** 2, axis=1, keepdims=True)
    y = (x - mu) * lax.rsqrt(var + 1e-5) * ln_s_ref[...] + ln_b_ref[...]
    h = jax.nn.gelu(jnp.dot(y, w1_ref[...], precision=lax.Precision.HIGHEST)
                    + b1_ref[...])
    out_ref[...] = x + jnp.dot(h, w2_ref[...],
                               precision=lax.Precision.HIGHEST) + b2_ref[...]


def kernel(queries, query_pos_encoding, query_bijl_indices,
           query_normalized_xy_positions, batch_offsets, stacked_feature_maps,
           spatial_shapes, token_predicted_salience, token_electron_probs,
           ln1_scale, ln1_bias, Wq, Wk, Wv, Wo, ln2_scale, ln2_bias,
           W1, b1, W2, b2):
    sal = token_predicted_salience.reshape(B, RT, 128)
    prob = token_electron_probs.reshape(B, RT, 128)

    pm = pl.pallas_call(
        _topk_positions_kernel,
        out_shape=jax.ShapeDtypeStruct((B, RT, 128), jnp.int32),
    )(sal, prob)
    pm3 = pm.reshape(B, 1, L)

    q3 = queries.reshape(B, L, D)
    pos3 = query_pos_encoding.reshape(B, L, D)

    full2 = lambda b: (0, 0)
    full1 = lambda b: (0,)
    new_q = pl.pallas_call(
        _attn_kernel,
        grid=(B,),
        in_specs=[
            pl.BlockSpec((1, 1, L), lambda b: (b, 0, 0)),
            pl.BlockSpec((1, L, D), lambda b: (b, 0, 0)),
            pl.BlockSpec((1, L, D), lambda b: (b, 0, 0)),
            pl.BlockSpec((D,), full1),
            pl.BlockSpec((D,), full1),
            pl.BlockSpec((D, D), full2),
            pl.BlockSpec((D, D), full2),
            pl.BlockSpec((D, D), full2),
            pl.BlockSpec((D, D), full2),
        ],
        out_specs=pl.BlockSpec((1, L, D), lambda b: (b, 0, 0)),
        out_shape=jax.ShapeDtypeStruct((B, L, D), jnp.float32),
    )(pm3, q3, pos3, ln1_scale, ln1_bias, Wq, Wk, Wv, Wo)

    xr = new_q.reshape(B * L, D)
    RTILE = 512
    out = pl.pallas_call(
        _ffn_kernel,
        grid=(B * L // RTILE,),
        in_specs=[
            pl.BlockSpec((RTILE, D), lambda r: (r, 0)),
            pl.BlockSpec((D,), full1),
            pl.BlockSpec((D,), full1),
            pl.BlockSpec((D, FF), full2),
            pl.BlockSpec((FF,), full1),
            pl.BlockSpec((FF, D), full2),
            pl.BlockSpec((D,), full1),
        ],
        out_specs=pl.BlockSpec((RTILE, D), lambda r: (r, 0)),
        out_shape=jax.ShapeDtypeStruct((B * L, D), jnp.float32),
    )(xr, ln2_scale, ln2_bias, W1, b1, W2, b2)
    return out


# speedup vs baseline: 4.7003x; 4.7003x over previous
import functools

import jax
import jax.numpy as jnp
from jax import lax
from jax.experimental import pallas as pl

B = 4
L = 4096
D = 256
H = 8
DH = D // H
FF = 1024
TOPK = 1000
KP = 1024
RT = 32
KT = 512
NKT = L // KT


def _topk_positions_kernel(sal_ref, prob_ref, pm_ref):
    s = prob_ref[...] * jax.nn.sigmoid(sal_ref[...])
    bits = lax.bitcast_convert_type(s, jnp.int32)
    lo0 = jnp.zeros((B, 1, 1), jnp.int32)
    hi0 = jnp.full((B, 1, 1), 0x7F7FFFFF, jnp.int32)

    def body(_, carry):
        lo, hi = carry
        mid = lo + (hi - lo + 1) // 2
        cnt = jnp.sum((bits >= mid).astype(jnp.int32), axis=(1, 2), keepdims=True)
        ok = cnt >= TOPK
        return jnp.where(ok, mid, lo), jnp.where(ok, hi, mid - 1)

    lo, _ = lax.fori_loop(0, 31, body, (lo0, hi0))
    mask = bits >= lo
    m = mask.astype(jnp.float32)
    c0 = lax.broadcasted_iota(jnp.int32, (128, 128), 0)
    c1 = lax.broadcasted_iota(jnp.int32, (128, 128), 1)
    upper = (c0 <= c1).astype(jnp.float32)
    r0 = lax.broadcasted_iota(jnp.int32, (RT, RT), 0)
    r1 = lax.broadcasted_iota(jnp.int32, (RT, RT), 1)
    strict = (r0 < r1).astype(jnp.float32)
    pin = lax.dot_general(m, upper, (((2,), (0,)), ((), ())),
                          precision=lax.Precision.HIGHEST)
    rs = jnp.sum(m, axis=2)
    ro = lax.dot_general(rs, strict, (((1,), (0,)), ((), ())),
                         precision=lax.Precision.HIGHEST)
    p = pin + ro[:, :, None]
    pm_ref[...] = jnp.where(mask, p.astype(jnp.int32), 0)


def _attn_kernel(pm_ref, q_ref, pos_ref, ln_s_ref, ln_b_ref,
                 wq_ref, wk_ref, wv_ref, wo_ref, out_ref):
    pmv = pm_ref[0, 0, :]
    hp = None

    selq = jnp.zeros((KP, D), jnp.float32)
    selpos = jnp.zeros((KP, D), jnp.float32)
    jr = lax.broadcasted_iota(jnp.int32, (KP, KT), 0) + 1
    for t in range(NKT):
        pm_t = pmv[t * KT:(t + 1) * KT]
        s_t = (pm_t[None, :] == jr).astype(jnp.float32)
        selq = selq + jnp.dot(s_t, q_ref[0, t * KT:(t + 1) * KT, :],
                              precision=hp)
        selpos = selpos + jnp.dot(s_t, pos_ref[0, t * KT:(t + 1) * KT, :],
                                  precision=hp)

    x = selq
    mu = jnp.mean(x, axis=1, keepdims=True)
    var = jnp.mean((x - mu) ** 2, axis=1, keepdims=True)
    x2 = (x - mu) * lax.rsqrt(var + 1e-5) * ln_s_ref[...] + ln_b_ref[...]
    qk_in = x2 + selpos
    q = jnp.dot(qk_in, wq_ref[...], precision=hp)
    k = jnp.dot(qk_in, wk_ref[...], precision=hp)
    v = jnp.dot(x2, wv_ref[...], precision=hp)

    scale = 1.0 / (DH ** 0.5)
    kcol = lax.broadcasted_iota(jnp.int32, (KP, KP), 1)
    kmask = jnp.where(kcol >= TOPK, -1e9, 0.0)
    ohs = []
    for h in range(H):
        qh = q[:, h * DH:(h + 1) * DH]
        kh = k[:, h * DH:(h + 1) * DH]
        vh = v[:, h * DH:(h + 1) * DH]
        logits = lax.dot_general(qh, kh, (((1,), (1,)), ((), ())),
                                 precision=hp) * scale + kmask
        mx = jnp.max(logits, axis=1, keepdims=True)
        e = jnp.exp(logits - mx)
        a = e / jnp.sum(e, axis=1, keepdims=True)
        ohs.append(jnp.dot(a, vh, precision=hp))
    oh = jnp.concatenate(ohs, axis=1)
    attn = x + jnp.dot(oh, wo_ref[...], precision=hp)

    for t in range(NKT):
        pm_t = pmv[t * KT:(t + 1) * KT]
        s_t = (pm_t[None, :] == jr).astype(jnp.float32)
        contrib = lax.dot_general(s_t, attn, (((0,), (0,)), ((), ())),
                                  precision=hp)
        keep = (pm_t == 0).astype(jnp.float32)[:, None]
        out_ref[0, t * KT:(t + 1) * KT, :] = (
            q_ref[0, t * KT:(t + 1) * KT, :] * keep + contrib)


def _ffn_kernel(x_ref, ln_s_ref, ln_b_ref, w1_ref, b1_ref, w2_ref, b2_ref,
                out_ref):
    x = x_ref[...]
    mu = jnp.mean(x, axis=1, keepdims=True)
    var = jnp.mean((x - mu) ** 2, axis=1, keepdims=True)
    y = (x - mu) * lax.rsqrt(var + 1e-5) * ln_s_ref[...] + ln_b_ref[...]
    h = jax.nn.gelu(jnp.dot(y, w1_ref[...]) + b1_ref[...])
    out_ref[...] = x + jnp.dot(h, w2_ref[...]) + b2_ref[...]


def kernel(queries, query_pos_encoding, query_bijl_indices,
           query_normalized_xy_positions, batch_offsets, stacked_feature_maps,
           spatial_shapes, token_predicted_salience, token_electron_probs,
           ln1_scale, ln1_bias, Wq, Wk, Wv, Wo, ln2_scale, ln2_bias,
           W1, b1, W2, b2):
    sal = token_predicted_salience.reshape(B, RT, 128)
    prob = token_electron_probs.reshape(B, RT, 128)

    pm = pl.pallas_call(
        _topk_positions_kernel,
        out_shape=jax.ShapeDtypeStruct((B, RT, 128), jnp.int32),
    )(sal, prob)
    pm3 = pm.reshape(B, 1, L)

    q3 = queries.reshape(B, L, D)
    pos3 = query_pos_encoding.reshape(B, L, D)

    full2 = lambda b: (0, 0)
    full1 = lambda b: (0,)
    new_q = pl.pallas_call(
        _attn_kernel,
        grid=(B,),
        in_specs=[
            pl.BlockSpec((1, 1, L), lambda b: (b, 0, 0)),
            pl.BlockSpec((1, L, D), lambda b: (b, 0, 0)),
            pl.BlockSpec((1, L, D), lambda b: (b, 0, 0)),
            pl.BlockSpec((D,), full1),
            pl.BlockSpec((D,), full1),
            pl.BlockSpec((D, D), full2),
            pl.BlockSpec((D, D), full2),
            pl.BlockSpec((D, D), full2),
            pl.BlockSpec((D, D), full2),
        ],
        out_specs=pl.BlockSpec((1, L, D), lambda b: (b, 0, 0)),
        out_shape=jax.ShapeDtypeStruct((B, L, D), jnp.float32),
    )(pm3, q3, pos3, ln1_scale, ln1_bias, Wq, Wk, Wv, Wo)

    xr = new_q.reshape(B * L, D)
    RTILE = 512
    out = pl.pallas_call(
        _ffn_kernel,
        grid=(B * L // RTILE,),
        in_specs=[
            pl.BlockSpec((RTILE, D), lambda r: (r, 0)),
            pl.BlockSpec((D,), full1),
            pl.BlockSpec((D,), full1),
            pl.BlockSpec((D, FF), full2),
            pl.BlockSpec((FF,), full1),
            pl.BlockSpec((FF, D), full2),
            pl.BlockSpec((D,), full1),
        ],
        out_specs=pl.BlockSpec((RTILE, D), lambda r: (r, 0)),
        out_shape=jax.ShapeDtypeStruct((B * L, D), jnp.float32),
    )(xr, ln2_scale, ln2_bias, W1, b1, W2, b2)
    return out
